# baseline (speedup 1.0000x reference)
import functools

import jax
import jax.numpy as jnp
from jax import lax
from jax.experimental import pallas as pl
from jax.experimental.pallas import tpu as pltpu
from jax.experimental.pallas import tpu_sc as plsc

C, L, P, H, Dh = 100, 12, 20, 12, 64
SLAB = P * H * Dh
DSPL = 32
SLAB_ROWS = P * H * (Dh // DSPL)
NSUB = 16
RPW = SLAB_ROWS // NSUB
EPW = RPW * DSPL
NLANE = 16

_SMESH = plsc.ScalarSubcoreMesh(axis_name="c", num_cores=2)
_VMESH = plsc.VectorSubcoreMesh(core_axis_name="c", subcore_axis_name="s")


@functools.partial(
    pl.kernel,
    mesh=_SMESH,
    out_type=[
        jax.ShapeDtypeStruct((SLAB, C), jnp.float32),
        jax.ShapeDtypeStruct((SLAB, C), jnp.float32),
    ],
    scratch_types=[
        pltpu.SMEM((2 * NLANE,), jnp.int32),
    ],
    compiler_params=pltpu.CompilerParams(needs_layout_passes=False),
)
def _stage_layer(idx_hbm, key_hbm, value_hbm, k_st, v_st, idx_s):
    c = lax.axis_index("c")
    pltpu.sync_copy(idx_hbm, idx_s)
    lay = idx_s[NLANE]

    @pl.when(c == 0)
    def _():
        pltpu.sync_copy(key_hbm.at[lay], k_st)

    @pl.when(c == 1)
    def _():
        pltpu.sync_copy(value_hbm.at[lay], v_st)


@functools.partial(
    pl.kernel,
    mesh=_VMESH,
    out_type=[
        jax.ShapeDtypeStruct((SLAB,), jnp.float32),
        jax.ShapeDtypeStruct((SLAB,), jnp.float32),
    ],
    scratch_types=[
        pltpu.VMEM((2 * NLANE,), jnp.int32),
        pltpu.VMEM((EPW, C), jnp.float32),
        pltpu.VMEM((EPW,), jnp.float32),
    ],
    compiler_params=pltpu.CompilerParams(needs_layout_passes=False),
)
def _extract_lane(idx_hbm, k_st, v_st, k_out, v_out, idx_v, buf, out_v):
    c = lax.axis_index("c")
    s = lax.axis_index("s")
    pltpu.sync_copy(idx_hbm, idx_v)
    cls_vec = idx_v[pl.ds(0, NLANE)]
    iota = lax.iota(jnp.int32, NLANE)

    def extract(src):
        pltpu.sync_copy(src.at[pl.ds(s * EPW, EPW)], buf)

        @pl.loop(0, EPW // NLANE)
        def _(i):
            j = i * NLANE + iota
            vals = plsc.load_gather(buf, [j, cls_vec])
            out_v[pl.ds(pl.multiple_of(i * NLANE, NLANE), NLANE)] = vals

    @pl.when(c == 0)
    def _():
        extract(k_st)
        pltpu.sync_copy(out_v, k_out.at[pl.ds(s * EPW, EPW)])

    @pl.when(c == 1)
    def _():
        extract(v_st)
        pltpu.sync_copy(out_v, v_out.at[pl.ds(s * EPW, EPW)])


def kernel(key, value, class_idx, layer_idx):
    cls = jnp.asarray(class_idx, jnp.int32)
    lay = jnp.asarray(layer_idx, jnp.int32)
    idx = jnp.concatenate(
        [jnp.full((NLANE,), cls, jnp.int32), jnp.full((NLANE,), lay, jnp.int32)]
    )
    kt = jnp.transpose(key, (1, 2, 3, 4, 0)).reshape(L, SLAB, C)
    vt = jnp.transpose(value, (1, 2, 3, 4, 0)).reshape(L, SLAB, C)
    k_st, v_st = _stage_layer(idx, kt, vt)
    ko, vo = _extract_lane(idx, k_st, v_st)
    return ko.reshape(P, H, Dh), vo.reshape(P, H, Dh)


# speedup vs baseline: 16.0067x; 16.0067x over previous
import functools

import jax
import jax.numpy as jnp
from jax import lax
from jax.experimental import pallas as pl
from jax.experimental.pallas import tpu as pltpu
from jax.experimental.pallas import tpu_sc as plsc

C, L, P, H, Dh = 100, 12, 20, 12, 64
SLAB = P * H * Dh
NROWS = L * SLAB
NSUB = 16
EPW = SLAB // NSUB
NLANE = 16

_MESH = plsc.VectorSubcoreMesh(core_axis_name="c", subcore_axis_name="s")


@functools.partial(
    pl.kernel,
    mesh=_MESH,
    out_type=[
        jax.ShapeDtypeStruct((SLAB,), jnp.float32),
        jax.ShapeDtypeStruct((SLAB,), jnp.float32),
    ],
    scratch_types=[
        pltpu.VMEM((2 * NLANE,), jnp.int32),
        pltpu.VMEM((EPW, C), jnp.float32),
        pltpu.VMEM((EPW,), jnp.float32),
    ],
    compiler_params=pltpu.CompilerParams(needs_layout_passes=False),
)
def _gather_slab(idx_hbm, key_hbm, value_hbm, k_out, v_out, idx_v, buf, out_v):
    c = lax.axis_index("c")
    s = lax.axis_index("s")
    pltpu.sync_copy(idx_hbm, idx_v)
    cls_vec = idx_v[pl.ds(0, NLANE)]
    lay = jnp.max(idx_v[pl.ds(NLANE, NLANE)])
    iota = lax.iota(jnp.int32, NLANE)
    base = pl.multiple_of(lay * SLAB + s * EPW, 8)

    def extract(src):
        pltpu.sync_copy(src.at[pl.ds(base, EPW)], buf)

        @pl.loop(0, EPW // NLANE)
        def _(i):
            j = i * NLANE + iota
            vals = plsc.load_gather(buf, [j, cls_vec])
            out_v[pl.ds(pl.multiple_of(i * NLANE, NLANE), NLANE)] = vals

    @pl.when(c == 0)
    def _():
        extract(key_hbm)
        pltpu.sync_copy(out_v, k_out.at[pl.ds(s * EPW, EPW)])

    @pl.when(c == 1)
    def _():
        extract(value_hbm)
        pltpu.sync_copy(out_v, v_out.at[pl.ds(s * EPW, EPW)])


def kernel(key, value, class_idx, layer_idx):
    cls = jnp.asarray(class_idx, jnp.int32)
    lay = jnp.asarray(layer_idx, jnp.int32)
    idx = jnp.concatenate(
        [jnp.full((NLANE,), cls, jnp.int32), jnp.full((NLANE,), lay, jnp.int32)]
    )
    kt = jnp.transpose(key, (1, 2, 3, 4, 0)).reshape(NROWS, C)
    vt = jnp.transpose(value, (1, 2, 3, 4, 0)).reshape(NROWS, C)
    ko, vo = _gather_slab(idx, kt, vt)
    return ko.reshape(P, H, Dh), vo.reshape(P, H, Dh)
